# baseline (speedup 1.0000x reference)
import jax
import jax.numpy as jnp
from jax.experimental import pallas as pl
from jax.experimental.pallas import tpu as pltpu

N_OBJ = 64
ATTR_DIM = 512
H = 256
W = 256
K_OPS = 8
N_COLORS = 10
TEMP = 0.3
_STRIP = 64
_NSTRIPS = H // _STRIP


def _scalar_kernel(attr_ref, wsel_ref, bsel_ref, wparam_ref, bparam_ref,
                   g_ref, scal_ref):
    attr = attr_ref[...].astype(jnp.bfloat16)
    sel_logits = (jnp.dot(attr, wsel_ref[...].astype(jnp.bfloat16),
                          preferred_element_type=jnp.float32)
                  + bsel_ref[...]) / 0.3
    t = sel_logits + g_ref[...]
    q = t / TEMP
    q = q - jnp.max(q, axis=1, keepdims=True)
    e = jnp.exp(q)
    y = e / jnp.sum(e, axis=1, keepdims=True)
    ym = jnp.max(y, axis=1, keepdims=True)
    iota = jax.lax.broadcasted_iota(jnp.int32, (N_OBJ, K_OPS), 1)
    k = jnp.min(jnp.where(y == ym, iota, K_OPS), axis=1, keepdims=True)

    params = 5.0 * (jnp.dot(attr, wparam_ref[...].astype(jnp.bfloat16),
                            preferred_element_type=jnp.float32)
                    + bparam_ref[...])
    psel = jnp.zeros((N_OBJ, N_COLORS), jnp.float32)
    for kk in range(K_OPS):
        psel = psel + jnp.where(k == kk, 1.0, 0.0) * params[:, kk * N_COLORS:(kk + 1) * N_COLORS]

    q2 = psel / TEMP
    q2 = q2 - jnp.max(q2, axis=1, keepdims=True)
    e2 = jnp.exp(q2)
    sm = e2 / jnp.sum(e2, axis=1, keepdims=True)
    cols = jnp.sum(
        sm * jax.lax.broadcasted_iota(
            jnp.int32, (N_OBJ, N_COLORS), 1).astype(jnp.float32),
        axis=1, keepdims=True)

    sig0 = jax.nn.sigmoid(psel[:, 0:1])
    sig1 = jax.nn.sigmoid(psel[:, 1:2])
    sig2 = jax.nn.sigmoid(psel[:, 2:3])
    sigm = jax.nn.sigmoid(jnp.mean(psel, axis=1, keepdims=True))

    w = jnp.where
    A = w(k == 0, 0.0,
        w(k == 3, -1.0,
        w(k == 5, 0.5,
        w(k == 6, 0.0,
        w(k == 7, 0.5 + sigm, 1.0)))))
    B = w(k == 0, cols,
        w(k == 1, 9.0 * sig0,
        w(k == 2, -9.0 * sig1,
        w(k == 3, 9.0,
        w(k == 5, 0.5 * cols, 0.0)))))
    C = w(k == 6, 9.0, 0.0)
    S = w(k == 6, 2.0 * sig2, 0.0)
    scal_ref[...] = jnp.concatenate([A, B, C, S], axis=1)


def _canvas_kernel(scal_ref, canvas_ref, mask_ref, out_ref):
    c0 = canvas_ref[...].astype(jnp.float32).astype(jnp.bfloat16)

    def body(i, c):
        a = scal_ref[i, 0]
        b = scal_ref[i, 1]
        cc = scal_ref[i, 2]
        s = scal_ref[i, 3]

        def finish(cn):
            cn = cn.astype(jnp.bfloat16)
            cn = jnp.clip(cn, 0.0, 9.0)
            return jnp.where(mask_ref[i], cn, c)

        def br_identity(c):
            return c

        def br_affine(c):
            return finish(a * c.astype(jnp.float32) + b)

        def br_sigmoid(c):
            cf = c.astype(jnp.float32)
            return finish(a * cf + b + cc * jax.nn.sigmoid((cf - 4.5) * s))

        idx = jnp.where(cc != 0.0, 2,
                        jnp.where((a == 1.0) & (b == 0.0), 0, 1))
        return jax.lax.switch(idx, (br_identity, br_affine, br_sigmoid), c)

    out_ref[...] = jax.lax.fori_loop(0, N_OBJ, body, c0).astype(jnp.float32)


def kernel(canvas, attr_tensor, obj_masks, W_sel, b_sel, W_param, b_param):
    key = jax.random.key(42)
    u = jax.random.uniform(key, (N_OBJ, K_OPS), minval=1e-6, maxval=1.0 - 1e-6)
    g = -jnp.log(-jnp.log(u))

    scal = pl.pallas_call(
        _scalar_kernel,
        out_shape=jax.ShapeDtypeStruct((N_OBJ, 4), jnp.float32),
    )(attr_tensor, W_sel, b_sel.reshape(1, K_OPS), W_param,
      b_param.reshape(1, K_OPS * N_COLORS), g)

    out = pl.pallas_call(
        _canvas_kernel,
        grid=(_NSTRIPS,),
        in_specs=[
            pl.BlockSpec(memory_space=pltpu.SMEM),
            pl.BlockSpec((_STRIP, W), lambda i: (i, 0)),
            pl.BlockSpec((N_OBJ, _STRIP, W), lambda i: (0, i, 0)),
        ],
        out_specs=pl.BlockSpec((_STRIP, W), lambda i: (i, 0)),
        out_shape=jax.ShapeDtypeStruct((H, W), jnp.float32),
        compiler_params=pltpu.CompilerParams(
            dimension_semantics=("parallel",)),
    )(scal, canvas, obj_masks)
    return out


# speedup vs baseline: 1.0450x; 1.0450x over previous
import jax
import jax.numpy as jnp
from jax.experimental import pallas as pl
from jax.experimental.pallas import tpu as pltpu

N_OBJ = 64
ATTR_DIM = 512
H = 256
W = 256
K_OPS = 8
N_COLORS = 10
TEMP = 0.3
_STRIP = 64
_NSTRIPS = H // _STRIP


def _scalar_kernel(attr_ref, wsel_ref, bsel_ref, wparam_ref, bparam_ref,
                   g_ref, scal_ref):
    attr = attr_ref[...].astype(jnp.bfloat16)
    sel_logits = (jnp.dot(attr, wsel_ref[...].astype(jnp.bfloat16),
                          preferred_element_type=jnp.float32)
                  + bsel_ref[...]) / 0.3
    t = sel_logits + g_ref[...]
    q = t / TEMP
    q = q - jnp.max(q, axis=1, keepdims=True)
    e = jnp.exp(q)
    y = e / jnp.sum(e, axis=1, keepdims=True)
    ym = jnp.max(y, axis=1, keepdims=True)
    iota = jax.lax.broadcasted_iota(jnp.int32, (N_OBJ, K_OPS), 1)
    k = jnp.min(jnp.where(y == ym, iota, K_OPS), axis=1, keepdims=True)

    params = 5.0 * (jnp.dot(attr, wparam_ref[...].astype(jnp.bfloat16),
                            preferred_element_type=jnp.float32)
                    + bparam_ref[...])
    psel = jnp.zeros((N_OBJ, N_COLORS), jnp.float32)
    for kk in range(K_OPS):
        psel = psel + jnp.where(k == kk, 1.0, 0.0) * params[:, kk * N_COLORS:(kk + 1) * N_COLORS]

    q2 = psel / TEMP
    q2 = q2 - jnp.max(q2, axis=1, keepdims=True)
    e2 = jnp.exp(q2)
    sm = e2 / jnp.sum(e2, axis=1, keepdims=True)
    cols = jnp.sum(
        sm * jax.lax.broadcasted_iota(
            jnp.int32, (N_OBJ, N_COLORS), 1).astype(jnp.float32),
        axis=1, keepdims=True)

    sig0 = jax.nn.sigmoid(psel[:, 0:1])
    sig1 = jax.nn.sigmoid(psel[:, 1:2])
    sig2 = jax.nn.sigmoid(psel[:, 2:3])
    sigm = jax.nn.sigmoid(jnp.mean(psel, axis=1, keepdims=True))

    w = jnp.where
    A = w(k == 0, 0.0,
        w(k == 3, -1.0,
        w(k == 5, 0.5,
        w(k == 6, 0.0,
        w(k == 7, 0.5 + sigm, 1.0)))))
    B = w(k == 0, cols,
        w(k == 1, 9.0 * sig0,
        w(k == 2, -9.0 * sig1,
        w(k == 3, 9.0,
        w(k == 5, 0.5 * cols, 0.0)))))
    C = w(k == 6, 9.0, 0.0)
    S = w(k == 6, 2.0 * sig2, 0.0)
    scal_ref[...] = jnp.concatenate([A, B, C, S], axis=1)


def _canvas_kernel(scal_ref, canvas_ref, mask_ref, out_ref):
    c0 = canvas_ref[...].astype(jnp.float32).astype(jnp.bfloat16)

    def step(i, c):
        m = mask_ref[i]
        a = scal_ref[i, 0]
        b = scal_ref[i, 1]
        cc = scal_ref[i, 2]
        s = scal_ref[i, 3]
        cf = c.astype(jnp.float32)
        cn = (a * cf + b + cc * jax.nn.sigmoid((cf - 4.5) * s)).astype(jnp.bfloat16)
        cn = jnp.clip(cn, 0.0, 9.0)
        return jnp.where(m, cn, c)

    def body(j, c):
        i = j * 2
        return step(i + 1, step(i, c))

    out_ref[...] = jax.lax.fori_loop(0, N_OBJ // 2, body, c0).astype(jnp.float32)


def kernel(canvas, attr_tensor, obj_masks, W_sel, b_sel, W_param, b_param):
    key = jax.random.key(42)
    u = jax.random.uniform(key, (N_OBJ, K_OPS), minval=1e-6, maxval=1.0 - 1e-6)
    g = -jnp.log(-jnp.log(u))

    scal = pl.pallas_call(
        _scalar_kernel,
        out_shape=jax.ShapeDtypeStruct((N_OBJ, 4), jnp.float32),
    )(attr_tensor, W_sel, b_sel.reshape(1, K_OPS), W_param,
      b_param.reshape(1, K_OPS * N_COLORS), g)

    out = pl.pallas_call(
        _canvas_kernel,
        grid=(_NSTRIPS,),
        in_specs=[
            pl.BlockSpec(memory_space=pltpu.SMEM),
            pl.BlockSpec((_STRIP, W), lambda i: (i, 0)),
            pl.BlockSpec((N_OBJ, _STRIP, W), lambda i: (0, i, 0)),
        ],
        out_specs=pl.BlockSpec((_STRIP, W), lambda i: (i, 0)),
        out_shape=jax.ShapeDtypeStruct((H, W), jnp.float32),
        compiler_params=pltpu.CompilerParams(
            dimension_semantics=("parallel",)),
    )(scal, canvas, obj_masks)
    return out
